# Initial kernel scaffold; baseline (speedup 1.0000x reference)
import jax
import jax.numpy as jnp
from jax.experimental import pallas as pl


def kernel(triplets, objects, latent, obj_emb, pred_emb, W1a, b1a, W1b, b1b, W2a, b2a, W2b, b2b, Wproj, bproj, proj2, logit_scale):
    raise NotImplementedError("write your pallas kernel here")


# speedup vs baseline: 10.1302x; 10.1302x over previous
import jax
import jax.numpy as jnp
from jax.experimental import pallas as pl
from jax.experimental.pallas import tpu as pltpu

_B = 64
_T = 10
_MAXO = 8
_NOBJ = 64
_NPRED = 32
_E = 1024
_H = 1024
_NL = 6
_NS = 8
_NE = _B * _T
_NO = _B * _MAXO

_F32 = jnp.float32


def _dot(a, b):
    return jax.lax.dot_general(a, b, (((1,), (0,)), ((), ())),
                               preferred_element_type=_F32)


def _dot_t(a, b):
    return jax.lax.dot_general(a, b, (((0,), (0,)), ((), ())),
                               preferred_element_type=_F32)


def _gnn_kernel(gs_ref, go_ref, obji_ref, pi_ref, w2t_ref, lat_ref, lsc_ref,
                obj_emb_ref, pred_emb_ref,
                W1a_ref, b1a_ref, W1b_ref, b1b_ref,
                W2a_ref, b2a_ref, W2b_ref, b2b_ref,
                Wproj_ref, bproj_ref,
                out1_ref, out2_ref,
                ov_s, pv_s, h_s, pool_s):
    l = pl.program_id(0)
    t = pl.program_id(1)

    @pl.when((l == 0) & (t == 0))
    def _init():
        oio = jax.lax.broadcasted_iota(jnp.int32, (_NO, _NOBJ + 1), 1)
        g_obj = (oio == obji_ref[...]).astype(_F32)
        ov_s[...] = _dot(g_obj, obj_emb_ref[...])
        pio = jax.lax.broadcasted_iota(jnp.int32, (_NE, _NPRED), 1)
        g_p = (pio == pi_ref[...]).astype(_F32)
        pv_s[...] = _dot(g_p, pred_emb_ref[...])

    eio = jax.lax.broadcasted_iota(jnp.int32, (_NE, _NO), 1)
    g_s = (eio == gs_ref[...]).astype(_F32)
    g_o = (eio == go_ref[...]).astype(_F32)

    @pl.when(t == 0)
    def _h0():
        h_s[...] = _dot(_dot(g_s, ov_s[...]), W1a_ref[0])

    @pl.when(t == 1)
    def _h1():
        h_s[...] = h_s[...] + _dot(pv_s[...], W1a_ref[0])

    @pl.when(t == 2)
    def _h2():
        acc = h_s[...] + _dot(_dot(g_o, ov_s[...]), W1a_ref[0])
        h_s[...] = jnp.maximum(acc + b1a_ref[0], 0.0)

    @pl.when((t == 3) & (l > 0))
    def _news():
        new_s = jnp.maximum(_dot(h_s[...], W1b_ref[0])
                            + b1b_ref[0, :, 0:_H], 0.0)
        pool_s[...] = _dot_t(g_s, new_s)

    @pl.when(t == 4)
    def _newp():
        pv_s[...] = jnp.maximum(_dot(h_s[...], W1b_ref[0])
                                + b1b_ref[0, :, _H:_H + _E], 0.0)

    @pl.when((t == 5) & (l > 0))
    def _newo():
        new_o = jnp.maximum(_dot(h_s[...], W1b_ref[0])
                            + b1b_ref[0, :, _H + _E:], 0.0)
        pooled = pool_s[...] + _dot_t(g_o, new_o)
        cnt = _dot_t(g_s + g_o, jnp.ones((_NE, 1), _F32))
        pool_s[...] = pooled * (1.0 / jnp.maximum(cnt, 1.0))

    @pl.when((t == 6) & (l > 0))
    def _h2obj():
        h_s[0:_NO, :] = jnp.maximum(_dot(pool_s[...], W2a_ref[0])
                                    + b2a_ref[0], 0.0)

    @pl.when((t == 7) & (l > 0))
    def _newobj():
        ov_s[...] = jnp.maximum(_dot(h_s[0:_NO, :], W2b_ref[0])
                                + b2b_ref[0], 0.0)

    @pl.when((t == 7) & (l == _NL - 1))
    def _final():
        g = _dot(ov_s[...], Wproj_ref[...]) + bproj_ref[...]
        bio = jax.lax.broadcasted_iota(jnp.int32, (_B, _NO), 0)
        rio = jax.lax.broadcasted_iota(jnp.int32, (_B, _NO), 1)
        pmat = jnp.where((rio >> 3) == bio,
                         jnp.broadcast_to(w2t_ref[...], (_B, _NO)), 0.0)
        gr = _dot(pmat, g)
        img = lat_ref[...]
        img_n = img * jax.lax.rsqrt(jnp.sum(img * img, axis=1, keepdims=True))
        gr_n = gr * jax.lax.rsqrt(jnp.sum(gr * gr, axis=1, keepdims=True))
        sc = jnp.exp(lsc_ref[...])
        out1_ref[...] = sc * jax.lax.dot_general(
            img_n, gr_n, (((1,), (1,)), ((), ())), preferred_element_type=_F32)
        out2_ref[...] = sc * jax.lax.dot_general(
            gr_n, img_n, (((1,), (1,)), ((), ())), preferred_element_type=_F32)


def _full(shape):
    nd = len(shape)
    return pl.BlockSpec(shape, lambda l, t, _nd=nd: (0,) * _nd)


def kernel(triplets, objects, latent, obj_emb, pred_emb,
           W1a, b1a, W1b, b1b, W2a, b2a, W2b, b2b,
           Wproj, bproj, proj2, logit_scale):
    s = triplets[:, :, 0].astype(jnp.int32)
    p = triplets[:, :, 1].astype(jnp.int32)
    o = triplets[:, :, 2].astype(jnp.int32)
    base = (jnp.arange(_B, dtype=jnp.int32) * _MAXO)[:, None]
    gs = (base + s).reshape(_NE, 1)
    go = (base + o).reshape(_NE, 1)
    pi = p.reshape(_NE, 1)
    obji = objects.astype(jnp.int32).reshape(_NO, 1)
    w2t = jnp.tile(proj2.reshape(1, _MAXO), (1, _B))
    lat2 = latent.reshape(_B, 32 * 32)
    lsc = logit_scale.reshape(1, 1)
    b1a3 = b1a.reshape(_NL, 1, _H)
    b1b3 = b1b.reshape(_NL, 1, 2 * _H + _E)
    b2a3 = b2a.reshape(_NL, 1, _H)
    b2b3 = b2b.reshape(_NL, 1, _E)
    bproj2 = bproj.reshape(1, _E)

    w1a_spec = pl.BlockSpec((1, _E, _H),
                            lambda l, t: (l, jnp.minimum(t, 2), 0))
    w1b_spec = pl.BlockSpec((1, _H, _E),
                            lambda l, t: (l, 0, jnp.clip(t - 3, 0, 2)))
    w2a_spec = pl.BlockSpec(
        (1, _H, _H),
        lambda l, t: (jnp.where(t >= 6, l, jnp.maximum(l - 1, 0)), 0, 0))
    w2b_spec = pl.BlockSpec(
        (1, _H, _E),
        lambda l, t: (jnp.where(t >= 7, l, jnp.maximum(l - 1, 0)), 0, 0))

    def _row(shape):
        return pl.BlockSpec((1,) + shape[1:], lambda l, t: (l, 0, 0))

    out_shape = (jax.ShapeDtypeStruct((_B, _B), _F32),
                 jax.ShapeDtypeStruct((_B, _B), _F32))
    in_specs = [
        _full((_NE, 1)), _full((_NE, 1)), _full((_NO, 1)), _full((_NE, 1)),
        _full((1, _NO)), _full((_B, _E)), _full((1, 1)),
        _full((_NOBJ + 1, _E)), _full((_NPRED, _E)),
        w1a_spec, _row((_NL, 1, _H)),
        w1b_spec, _row((_NL, 1, 2 * _H + _E)),
        w2a_spec, _row((_NL, 1, _H)),
        w2b_spec, _row((_NL, 1, _E)),
        _full((_E, _E)), _full((1, _E)),
    ]
    out_specs = (_full((_B, _B)), _full((_B, _B)))
    fn = pl.pallas_call(
        _gnn_kernel,
        grid=(_NL, _NS),
        in_specs=in_specs,
        out_specs=out_specs,
        out_shape=out_shape,
        scratch_shapes=[pltpu.VMEM((_NO, _E), _F32),
                        pltpu.VMEM((_NE, _E), _F32),
                        pltpu.VMEM((_NE, _H), _F32),
                        pltpu.VMEM((_NO, _H), _F32)],
        compiler_params=pltpu.CompilerParams(
            dimension_semantics=("arbitrary", "arbitrary")),
    )
    return fn(gs, go, obji, pi, w2t, lat2, lsc, obj_emb, pred_emb,
              W1a, b1a3, W1b, b1b3, W2a, b2a3, W2b, b2b3, Wproj, bproj2)
